# Initial kernel scaffold; baseline (speedup 1.0000x reference)
import jax
import jax.numpy as jnp
from jax.experimental import pallas as pl


def kernel(f0, emb_f0, f1, emb_f1, f2, emb_f2, f3, emb_f3, f4, emb_f4, f5, emb_f5, f6, emb_f6, f7, emb_f7, f8, emb_f8, f9, emb_f9, f10, emb_f10, f11, emb_f11, f12, emb_f12, f13, emb_f13, f14, emb_f14, f15, emb_f15, f16, emb_f16, f17, emb_f17, f18, emb_f18, f19, emb_f19, f20, emb_f20, f21, emb_f21, f22, emb_f22, f23, emb_f23, f24, emb_f24, f25, emb_f25, W1, b1, W2, b2):
    raise NotImplementedError("write your pallas kernel here")


# speedup vs baseline: 11.9115x; 11.9115x over previous
import jax
import jax.numpy as jnp
from jax.experimental import pallas as pl
from jax.experimental.pallas import tpu as pltpu

N_FEAT = 26
VOCAB = 120
VOCAB_PAD = 128
EMB = 32
BATCH = 16384
HID = 256
OUT_DIM = 64
TILE_B = 512


def _fused_body(idx_ref, tab_ref, w1_ref, b1_ref, w2_ref, b2_ref, out_ref):
    lane = jax.lax.broadcasted_iota(jnp.int32, (TILE_B, VOCAB_PAD), 1)
    pieces = []
    for i in range(N_FEAT):
        idx_i = idx_ref[0, :, i]
        oh = (idx_i[:, None] == lane).astype(jnp.float32)
        pieces.append(jax.lax.dot(oh, tab_ref[i],
                                  preferred_element_type=jnp.float32))
    x = jnp.concatenate(pieces, axis=1)
    h = jax.lax.dot(x, w1_ref[...], preferred_element_type=jnp.float32)
    h = jnp.maximum(h + b1_ref[...], 0.0)
    z = jax.lax.dot(h, w2_ref[...], preferred_element_type=jnp.float32)
    z = z + b2_ref[...]
    n = jnp.sqrt(jnp.sum(z * z, axis=1, keepdims=True))
    out_ref[...] = z / jnp.maximum(n, 1e-12)


def kernel(f0, emb_f0, f1, emb_f1, f2, emb_f2, f3, emb_f3, f4, emb_f4,
           f5, emb_f5, f6, emb_f6, f7, emb_f7, f8, emb_f8, f9, emb_f9,
           f10, emb_f10, f11, emb_f11, f12, emb_f12, f13, emb_f13,
           f14, emb_f14, f15, emb_f15, f16, emb_f16, f17, emb_f17,
           f18, emb_f18, f19, emb_f19, f20, emb_f20, f21, emb_f21,
           f22, emb_f22, f23, emb_f23, f24, emb_f24, f25, emb_f25,
           W1, b1, W2, b2):
    feats = [f0, f1, f2, f3, f4, f5, f6, f7, f8, f9, f10, f11, f12,
             f13, f14, f15, f16, f17, f18, f19, f20, f21, f22, f23,
             f24, f25]
    tabs = [emb_f0, emb_f1, emb_f2, emb_f3, emb_f4, emb_f5, emb_f6,
            emb_f7, emb_f8, emb_f9, emb_f10, emb_f11, emb_f12, emb_f13,
            emb_f14, emb_f15, emb_f16, emb_f17, emb_f18, emb_f19,
            emb_f20, emb_f21, emb_f22, emb_f23, emb_f24, emb_f25]
    idx = jnp.stack([f.astype(jnp.int32) for f in feats], axis=1)
    idx = idx.reshape(1, BATCH, N_FEAT)
    tab = jnp.stack(tabs, axis=0)
    tab = jnp.pad(tab, ((0, 0), (0, VOCAB_PAD - VOCAB), (0, 0)))

    grid = BATCH // TILE_B
    return pl.pallas_call(
        _fused_body,
        grid=(grid,),
        in_specs=[
            pl.BlockSpec((1, TILE_B, N_FEAT), lambda i: (0, i, 0)),
            pl.BlockSpec((N_FEAT, VOCAB_PAD, EMB), lambda i: (0, 0, 0)),
            pl.BlockSpec((N_FEAT * EMB, HID), lambda i: (0, 0)),
            pl.BlockSpec((HID,), lambda i: (0,)),
            pl.BlockSpec((HID, OUT_DIM), lambda i: (0, 0)),
            pl.BlockSpec((OUT_DIM,), lambda i: (0,)),
        ],
        out_specs=pl.BlockSpec((TILE_B, OUT_DIM), lambda i: (i, 0)),
        out_shape=jax.ShapeDtypeStruct((BATCH, OUT_DIM), jnp.float32),
    )(idx, tab, W1, b1, W2, b2)
